# baseline (speedup 1.0000x reference)
import functools

import numpy as np
import jax
import jax.numpy as jnp
from jax import lax
from jax.experimental import pallas as pl
from jax.experimental.pallas import tpu as pltpu
from jax.experimental.pallas import tpu_sc as plsc

EMB = 128
D0, D1, D2 = 5, 6, 2
NCODE = D0 * D1 * D2
NCODE_PAD = 64
KPAD = 16

NC, NS, LANES = 2, 16, 16
NW = NC * NS

REG = 400
GROUPS = REG // LANES


def _onehot_const() -> np.ndarray:
    o = np.zeros((NCODE_PAD, KPAD), dtype=np.float32)
    for i0 in range(D0):
        for i1 in range(D1):
            for i2 in range(D2):
                c = i0 * (D1 * D2) + i1 * D2 + i2
                o[c, i0] += 1.0
                o[c, D0 + i1] += 1.0
                o[c, D0 + D1 + i2] += 1.0
    return o


_ONEHOT = _onehot_const()


def _combo_body(o_ref, w_ref, c_ref):
    c_ref[...] = jnp.dot(o_ref[...], w_ref[...],
                         preferred_element_type=jnp.float32)


def _build_combo_table(w_cat):
    return pl.pallas_call(
        _combo_body,
        grid=(NW,),
        in_specs=[
            pl.BlockSpec((NCODE_PAD, KPAD), lambda i: (0, 0)),
            pl.BlockSpec((KPAD, EMB), lambda i: (0, 0)),
        ],
        out_specs=pl.BlockSpec((NCODE_PAD, EMB), lambda i: (i, 0)),
        out_shape=jax.ShapeDtypeStruct((NW * NCODE_PAD, EMB), jnp.float32),
    )(jnp.asarray(_ONEHOT), w_cat)


def _sc_body(n_edges, per_w, a0_hbm, a1_hbm, a2_hbm, ctab_hbm, out_hbm,
             ctab_v, aw, winbuf, asems, wsems, isem):
    wid = lax.axis_index("s") * NC + lax.axis_index("c")
    base = wid * per_w
    nwin = per_w // REG

    cp = pltpu.make_async_copy(ctab_hbm.at[wid], ctab_v, isem)
    cp.start()

    def a_fetch(w, b):
        off = base + w * REG
        return (pltpu.make_async_copy(a0_hbm.at[pl.ds(off, REG)],
                                      aw[b][0], asems[b]),
                pltpu.make_async_copy(a1_hbm.at[pl.ds(off, REG)],
                                      aw[b][1], asems[b]),
                pltpu.make_async_copy(a2_hbm.at[pl.ds(off, REG)],
                                      aw[b][2], asems[b]))

    def a_start(w, b):
        for c in a_fetch(w, b):
            c.start()

    def a_wait(w, b):
        for c in a_fetch(w, b):
            c.wait()

    def write(w, b):
        return pltpu.make_async_copy(
            winbuf[b],
            out_hbm.at[pl.ds((base + w * REG) * EMB, REG * EMB)],
            wsems[b])

    iot = lax.iota(jnp.int32, LANES)
    cols = [iot + j * LANES for j in range(EMB // LANES)]

    def compose(w, b):
        def group(g, _):
            s16 = pl.ds(g * LANES, LANES)
            code16 = (aw[b][0][s16] * (D1 * D2) + aw[b][1][s16] * D2
                      + aw[b][2][s16]) * EMB
            row0 = g * (LANES * EMB)
            for e in range(LANES):
                fbase = code16.at[jnp.full((LANES,), e, jnp.int32)].get(
                    mode="promise_in_bounds")
                vs = [plsc.load_gather(ctab_v, [fbase + cols[j]])
                      for j in range(EMB // LANES)]
                for j, v in enumerate(vs):
                    winbuf[b][pl.ds(row0 + e * EMB + j * LANES, LANES)] = v
            return 0
        lax.fori_loop(0, GROUPS, group, 0)

    a_start(0, 0)
    a_start(1, 1)
    cp.wait()

    a_wait(0, 0)
    compose(0, 0)
    write(0, 0).start()
    a_start(2, 0)

    assert (nwin - 1) % 2 == 0

    def pair_body(ww, _):
        for step in range(2):
            w = 1 + ww * 2 + step
            b = (1 + step) % 2
            a_wait(w, b)

            @pl.when(w >= 2)
            def _():
                write(w - 2, b).wait()
            compose(w, b)
            write(w, b).start()

            @pl.when(w + 2 <= nwin - 1)
            def _():
                a_start(w + 2, b)
        return 0

    lax.fori_loop(0, (nwin - 1) // 2, pair_body, 0)

    write(nwin - 2, (nwin - 2) % 2).wait()
    write(nwin - 1, (nwin - 1) % 2).wait()


def _edge_gather(a0, a1, a2, ctab, n_edges):
    per_w = n_edges // NW
    mesh = plsc.VectorSubcoreMesh(core_axis_name="c", subcore_axis_name="s")
    return pl.kernel(
        functools.partial(_sc_body, n_edges, per_w),
        out_type=jax.ShapeDtypeStruct((n_edges * EMB,), jnp.float32),
        mesh=mesh,
        compiler_params=pltpu.CompilerParams(needs_layout_passes=False),
        scratch_types=[
            pltpu.VMEM((NCODE_PAD * EMB,), jnp.float32),
            tuple(tuple(pltpu.VMEM((REG,), jnp.int32) for _ in range(3))
                  for _ in range(2)),
            tuple(pltpu.VMEM((REG * EMB,), jnp.float32) for _ in range(2)),
            tuple(pltpu.SemaphoreType.DMA for _ in range(2)),
            tuple(pltpu.SemaphoreType.DMA for _ in range(2)),
            pltpu.SemaphoreType.DMA,
        ],
    )(a0, a1, a2, ctab)


def kernel(edge_attr, W0, W1, W2):
    n_edges = edge_attr.shape[0]
    attr = edge_attr.astype(jnp.int32)
    a0 = attr[:, 0]
    a1 = attr[:, 1]
    a2 = attr[:, 2]
    w_cat = jnp.concatenate(
        [W0, W1, W2, jnp.zeros((KPAD - D0 - D1 - D2, EMB), jnp.float32)],
        axis=0)
    ctab = _build_combo_table(w_cat).reshape(NW, NCODE_PAD * EMB)
    out = _edge_gather(a0, a1, a2, ctab, n_edges)
    return out.reshape(n_edges, EMB)


# speedup vs baseline: 1.0115x; 1.0115x over previous
import functools

import jax
import jax.numpy as jnp
from jax import lax
from jax.experimental import pallas as pl
from jax.experimental.pallas import tpu as pltpu
from jax.experimental.pallas import tpu_sc as plsc

EMB = 128
D0, D1, D2 = 5, 6, 2
NCODE = D0 * D1 * D2
NCODE_PAD = 64
KPAD = 16

NC, NS, LANES = 2, 16, 16
NW = NC * NS

REG = 400
GROUPS = REG // LANES


def _combo_body(w_ref, c_ref):
    zero = jnp.zeros((EMB,), jnp.float32)
    for c in range(NCODE_PAD):
        if c < NCODE:
            i0, i1, i2 = c // (D1 * D2), (c // D2) % D1, c % D2
            c_ref[c] = (w_ref[i0] + w_ref[D0 + i1]) + w_ref[D0 + D1 + i2]
        else:
            c_ref[c] = zero


def _build_combo_table(w_cat):
    return pl.pallas_call(
        _combo_body,
        grid=(NW,),
        in_specs=[pl.BlockSpec((KPAD, EMB), lambda i: (0, 0))],
        out_specs=pl.BlockSpec((NCODE_PAD, EMB), lambda i: (i, 0)),
        out_shape=jax.ShapeDtypeStruct((NW * NCODE_PAD, EMB), jnp.float32),
    )(w_cat)


def _sc_body(n_edges, per_w, a0_hbm, a1_hbm, a2_hbm, ctab_hbm, out_hbm,
             ctab_v, aw, winbuf, asems, wsems, isem):
    wid = lax.axis_index("s") * NC + lax.axis_index("c")
    base = wid * per_w
    nwin = per_w // REG

    cp = pltpu.make_async_copy(ctab_hbm.at[wid], ctab_v, isem)
    cp.start()

    def a_fetch(w, b):
        off = base + w * REG
        return (pltpu.make_async_copy(a0_hbm.at[pl.ds(off, REG)],
                                      aw[b][0], asems[b]),
                pltpu.make_async_copy(a1_hbm.at[pl.ds(off, REG)],
                                      aw[b][1], asems[b]),
                pltpu.make_async_copy(a2_hbm.at[pl.ds(off, REG)],
                                      aw[b][2], asems[b]))

    def a_start(w, b):
        for c in a_fetch(w, b):
            c.start()

    def a_wait(w, b):
        for c in a_fetch(w, b):
            c.wait()

    def write(w, b):
        return pltpu.make_async_copy(
            winbuf[b],
            out_hbm.at[pl.ds((base + w * REG) * EMB, REG * EMB)],
            wsems[b])

    iot = lax.iota(jnp.int32, LANES)
    cols = [iot + j * LANES for j in range(EMB // LANES)]

    def compose(w, b):
        def group(g, _):
            s16 = pl.ds(g * LANES, LANES)
            code16 = (aw[b][0][s16] * (D1 * D2) + aw[b][1][s16] * D2
                      + aw[b][2][s16]) * EMB
            row0 = g * (LANES * EMB)
            for e in range(LANES):
                fbase = code16.at[jnp.full((LANES,), e, jnp.int32)].get(
                    mode="promise_in_bounds")
                vs = [plsc.load_gather(ctab_v, [fbase + cols[j]])
                      for j in range(EMB // LANES)]
                for j, v in enumerate(vs):
                    winbuf[b][pl.ds(row0 + e * EMB + j * LANES, LANES)] = v
            return 0
        lax.fori_loop(0, GROUPS, group, 0)

    a_start(0, 0)
    a_start(1, 1)
    cp.wait()

    a_wait(0, 0)
    compose(0, 0)
    write(0, 0).start()
    a_start(2, 0)

    assert (nwin - 1) % 2 == 0

    def pair_body(ww, _):
        for step in range(2):
            w = 1 + ww * 2 + step
            b = (1 + step) % 2
            a_wait(w, b)

            @pl.when(w >= 2)
            def _():
                write(w - 2, b).wait()
            compose(w, b)
            write(w, b).start()

            @pl.when(w + 2 <= nwin - 1)
            def _():
                a_start(w + 2, b)
        return 0

    lax.fori_loop(0, (nwin - 1) // 2, pair_body, 0)

    write(nwin - 2, (nwin - 2) % 2).wait()
    write(nwin - 1, (nwin - 1) % 2).wait()


def _edge_gather(a0, a1, a2, ctab, n_edges):
    per_w = n_edges // NW
    mesh = plsc.VectorSubcoreMesh(core_axis_name="c", subcore_axis_name="s")
    return pl.kernel(
        functools.partial(_sc_body, n_edges, per_w),
        out_type=jax.ShapeDtypeStruct((n_edges * EMB,), jnp.float32),
        mesh=mesh,
        compiler_params=pltpu.CompilerParams(needs_layout_passes=False),
        scratch_types=[
            pltpu.VMEM((NCODE_PAD * EMB,), jnp.float32),
            tuple(tuple(pltpu.VMEM((REG,), jnp.int32) for _ in range(3))
                  for _ in range(2)),
            tuple(pltpu.VMEM((REG * EMB,), jnp.float32) for _ in range(2)),
            tuple(pltpu.SemaphoreType.DMA for _ in range(2)),
            tuple(pltpu.SemaphoreType.DMA for _ in range(2)),
            pltpu.SemaphoreType.DMA,
        ],
    )(a0, a1, a2, ctab)


def kernel(edge_attr, W0, W1, W2):
    n_edges = edge_attr.shape[0]
    attr = edge_attr.astype(jnp.int32)
    a0 = attr[:, 0]
    a1 = attr[:, 1]
    a2 = attr[:, 2]
    w_cat = jnp.concatenate(
        [W0, W1, W2, jnp.zeros((KPAD - D0 - D1 - D2, EMB), jnp.float32)],
        axis=0)
    ctab = _build_combo_table(w_cat).reshape(NW, NCODE_PAD * EMB)
    out = _edge_gather(a0, a1, a2, ctab, n_edges)
    return out.reshape(n_edges, EMB)
